# Initial kernel scaffold; baseline (speedup 1.0000x reference)
import jax
import jax.numpy as jnp
from jax.experimental import pallas as pl


def kernel(x, node_params, edge_params, edge_index, volume_id):
    raise NotImplementedError("write your pallas kernel here")


# speedup vs baseline: 4.7549x; 4.7549x over previous
import jax
import jax.numpy as jnp
from jax import lax
from jax.experimental import pallas as pl
from jax.experimental.pallas import tpu as pltpu
from jax.experimental.pallas import tpu_sc as plsc

HIDDEN = 128
N_NODES = 10000
N_EDGES = 160000
NB_NODE = 1000
NB_EDGE = 800
CH = 128
NW = 32
N_CHUNKS = N_EDGES // CH
EPS = 1e-5


def _ln_norm(z):
    mu = jnp.mean(z, axis=-1, keepdims=True)
    d = z - mu
    var = jnp.mean(d * d, axis=-1, keepdims=True)
    return d / jnp.sqrt(var + EPS)


def _node_body(x_ref, vol_ref, wn1, bn1, gn1, en1, wn2, bn2, gn2, en2,
               wn3, bn3, gn3, en3, wt, bt, wb, enc_ref, p_ref, q_ref):
    x = x_ref[...]
    vol = vol_ref[...]
    encs = []
    for m in range(2):
        h = jnp.dot(x, wn1[m], preferred_element_type=jnp.float32) + bn1[m]
        h = jax.nn.relu(_ln_norm(h) * gn1[m] + en1[m])
        h = jnp.dot(h, wn2[m], preferred_element_type=jnp.float32) + bn2[m]
        h = jax.nn.relu(_ln_norm(h) * gn2[m] + en2[m])
        h = jnp.dot(h, wn3[m], preferred_element_type=jnp.float32) + bn3[m]
        h = jnp.tanh(_ln_norm(h) * gn3[m] + en3[m])
        encs.append(h)
    enc = jnp.where(vol == 1.0, encs[1], encs[0])
    enc_ref[...] = enc
    for e in range(3):
        p_ref[e, :, :] = jnp.dot(enc, wt[e], preferred_element_type=jnp.float32) + bt[e]
        q_ref[e, :, :] = jnp.dot(enc, wb[e], preferred_element_type=jnp.float32)


def _sc_body(p_hbm, q_hbm, s_hbm, e_hbm, vol_hbm, h1_hbm, eid_hbm,
             vol_v, sidx, eidx, rp, rq, eidv, prow, qrow, sem_p, sem_q):
    wid = lax.axis_index("s") * 2 + lax.axis_index("c")
    pltpu.sync_copy(vol_hbm, vol_v)
    nch = (N_CHUNKS // NW) + jnp.where(wid < (N_CHUNKS % NW), 1, 0)

    def chunk(it, carry):
        base = (wid + it * NW) * CH
        pltpu.sync_copy(s_hbm.at[pl.ds(base, CH)], sidx)
        pltpu.sync_copy(e_hbm.at[pl.ds(base, CH)], eidx)
        for g in range(CH // 16):
            sl = pl.ds(g * 16, 16)
            s16 = sidx[sl]
            e16 = eidx[sl]
            vs = plsc.load_gather(vol_v, [s16])
            ve = plsc.load_gather(vol_v, [e16])
            ex = vs + ve
            eid = jnp.where((vs == 1) & (ve == 0), 3, ex)
            eidv[sl] = eid.astype(jnp.float32)
            rp[sl] = ex * N_NODES + s16
            rq[sl] = ex * N_NODES + e16
        cp = pltpu.async_copy(p_hbm.at[rp], prow, sem_p)
        cq = pltpu.async_copy(q_hbm.at[rq], qrow, sem_q)
        cp.wait()
        cq.wait()

        def addrow(r, c2):
            for c in range(HIDDEN // 16):
                cs = pl.ds(c * 16, 16)
                prow[r, cs] = prow[r, cs] + qrow[r, cs]
            return c2
        lax.fori_loop(0, CH, addrow, 0)
        pltpu.sync_copy(prow, h1_hbm.at[pl.ds(base, CH)])
        pltpu.sync_copy(eidv, eid_hbm.at[pl.ds(base, CH)])
        return carry
    lax.fori_loop(0, nch, chunk, 0)


def _edge_body(h1_ref, eid_ref, g1, e1, w2, b2, g2, e2, w3, b3, g3, e3, out_ref):
    h1 = h1_ref[...]
    eid = eid_ref[...]
    xh = _ln_norm(h1)
    acc = jnp.zeros_like(h1)
    for e in range(3):
        a1 = jax.nn.relu(xh * g1[e] + e1[e])
        z2 = jnp.dot(a1, w2[e], preferred_element_type=jnp.float32) + b2[e]
        a2 = jax.nn.relu(_ln_norm(z2) * g2[e] + e2[e])
        z3 = jnp.dot(a2, w3[e], preferred_element_type=jnp.float32) + b3[e]
        y = jnp.tanh(_ln_norm(z3) * g3[e] + e3[e])
        acc = acc + jnp.where(eid == float(e), y, 0.0)
    out_ref[...] = acc


def _full(shape):
    return pl.BlockSpec(shape, lambda i: (0,) * len(shape))


def kernel(x, node_params, edge_params, edge_index, volume_id):
    f32 = jnp.float32
    w0 = jnp.concatenate([node_params[0][0][0], jnp.zeros((4, HIDDEN), f32)], axis=0)
    wn1 = jnp.stack([w0, node_params[1][0][0]])
    bn1 = jnp.stack([node_params[0][0][1], node_params[1][0][1]])
    gn1 = jnp.stack([node_params[0][0][2], node_params[1][0][2]])
    en1 = jnp.stack([node_params[0][0][3], node_params[1][0][3]])
    wn2 = jnp.stack([node_params[0][1][0], node_params[1][1][0]])
    bn2 = jnp.stack([node_params[0][1][1], node_params[1][1][1]])
    gn2 = jnp.stack([node_params[0][1][2], node_params[1][1][2]])
    en2 = jnp.stack([node_params[0][1][3], node_params[1][1][3]])
    wn3 = jnp.stack([node_params[0][2][0], node_params[1][2][0]])
    bn3 = jnp.stack([node_params[0][2][1], node_params[1][2][1]])
    gn3 = jnp.stack([node_params[0][2][2], node_params[1][2][2]])
    en3 = jnp.stack([node_params[0][2][3], node_params[1][2][3]])

    wt = jnp.stack([edge_params[e][0][0][:HIDDEN] for e in range(3)])
    wb = jnp.stack([edge_params[e][0][0][HIDDEN:] for e in range(3)])
    bt = jnp.stack([edge_params[e][0][1] for e in range(3)])
    g1 = jnp.stack([edge_params[e][0][2] for e in range(3)])
    e1 = jnp.stack([edge_params[e][0][3] for e in range(3)])
    w2 = jnp.stack([edge_params[e][1][0] for e in range(3)])
    b2 = jnp.stack([edge_params[e][1][1] for e in range(3)])
    g2 = jnp.stack([edge_params[e][1][2] for e in range(3)])
    e2 = jnp.stack([edge_params[e][1][3] for e in range(3)])
    w3 = jnp.stack([edge_params[e][2][0] for e in range(3)])
    b3 = jnp.stack([edge_params[e][2][1] for e in range(3)])
    g3 = jnp.stack([edge_params[e][2][2] for e in range(3)])
    e3 = jnp.stack([edge_params[e][2][3] for e in range(3)])

    volf = volume_id.astype(f32)[:, None]
    voli = volume_id.astype(jnp.int32)
    start = edge_index[0].astype(jnp.int32)
    end = edge_index[1].astype(jnp.int32)

    n_blocks = N_NODES // NB_NODE
    enc, p_tab, q_tab = pl.pallas_call(
        _node_body,
        grid=(n_blocks,),
        in_specs=[
            pl.BlockSpec((NB_NODE, 12), lambda i: (i, 0)),
            pl.BlockSpec((NB_NODE, 1), lambda i: (i, 0)),
            _full((2, 12, HIDDEN)), _full((2, HIDDEN)), _full((2, HIDDEN)), _full((2, HIDDEN)),
            _full((2, HIDDEN, HIDDEN)), _full((2, HIDDEN)), _full((2, HIDDEN)), _full((2, HIDDEN)),
            _full((2, HIDDEN, HIDDEN)), _full((2, HIDDEN)), _full((2, HIDDEN)), _full((2, HIDDEN)),
            _full((3, HIDDEN, HIDDEN)), _full((3, HIDDEN)), _full((3, HIDDEN, HIDDEN)),
        ],
        out_specs=[
            pl.BlockSpec((NB_NODE, HIDDEN), lambda i: (i, 0)),
            pl.BlockSpec((3, NB_NODE, HIDDEN), lambda i: (0, i, 0)),
            pl.BlockSpec((3, NB_NODE, HIDDEN), lambda i: (0, i, 0)),
        ],
        out_shape=[
            jax.ShapeDtypeStruct((N_NODES, HIDDEN), f32),
            jax.ShapeDtypeStruct((3, N_NODES, HIDDEN), f32),
            jax.ShapeDtypeStruct((3, N_NODES, HIDDEN), f32),
        ],
    )(x, volf, wn1, bn1, gn1, en1, wn2, bn2, gn2, en2, wn3, bn3, gn3, en3, wt, bt, wb)

    p_flat = p_tab.reshape(3 * N_NODES, HIDDEN)
    q_flat = q_tab.reshape(3 * N_NODES, HIDDEN)

    mesh = plsc.VectorSubcoreMesh(core_axis_name="c", subcore_axis_name="s")
    h1, eidf = pl.kernel(
        _sc_body,
        out_type=[
            jax.ShapeDtypeStruct((N_EDGES, HIDDEN), f32),
            jax.ShapeDtypeStruct((N_EDGES,), f32),
        ],
        mesh=mesh,
        compiler_params=pltpu.CompilerParams(needs_layout_passes=False),
        scratch_types=[
            pltpu.VMEM((N_NODES,), jnp.int32),
            pltpu.VMEM((CH,), jnp.int32),
            pltpu.VMEM((CH,), jnp.int32),
            pltpu.VMEM((CH,), jnp.int32),
            pltpu.VMEM((CH,), jnp.int32),
            pltpu.VMEM((CH,), f32),
            pltpu.VMEM((CH, HIDDEN), f32),
            pltpu.VMEM((CH, HIDDEN), f32),
            pltpu.SemaphoreType.DMA,
            pltpu.SemaphoreType.DMA,
        ],
    )(p_flat, q_flat, start, end, voli)

    e_blocks = N_EDGES // NB_EDGE
    out_edges = pl.pallas_call(
        _edge_body,
        grid=(e_blocks,),
        in_specs=[
            pl.BlockSpec((NB_EDGE, HIDDEN), lambda i: (i, 0)),
            pl.BlockSpec((NB_EDGE, 1), lambda i: (i, 0)),
            _full((3, HIDDEN)), _full((3, HIDDEN)),
            _full((3, HIDDEN, HIDDEN)), _full((3, HIDDEN)), _full((3, HIDDEN)), _full((3, HIDDEN)),
            _full((3, HIDDEN, HIDDEN)), _full((3, HIDDEN)), _full((3, HIDDEN)), _full((3, HIDDEN)),
        ],
        out_specs=pl.BlockSpec((NB_EDGE, HIDDEN), lambda i: (i, 0)),
        out_shape=jax.ShapeDtypeStruct((N_EDGES, HIDDEN), f32),
    )(h1, eidf[:, None], g1, e1, w2, b2, g2, e2, w3, b3, g3, e3)

    return (enc, out_edges)


# speedup vs baseline: 5.3706x; 1.1295x over previous
import jax
import jax.numpy as jnp
from jax import lax
from jax.experimental import pallas as pl
from jax.experimental.pallas import tpu as pltpu
from jax.experimental.pallas import tpu_sc as plsc

HIDDEN = 128
N_NODES = 10000
N_EDGES = 160000
NB_NODE = 1000
NB_EDGE = 800
CH = 128
NW = 32
N_CHUNKS = N_EDGES // CH
EPS = 1e-5


def _ln_norm(z):
    mu = jnp.mean(z, axis=-1, keepdims=True)
    d = z - mu
    var = jnp.mean(d * d, axis=-1, keepdims=True)
    return d / jnp.sqrt(var + EPS)


def _bdot(a, w):
    return jnp.dot(a.astype(jnp.bfloat16), w, preferred_element_type=jnp.float32)


def _node_body(x_ref, vol_ref, wn1, bn1, gn1, en1, wn2, bn2, gn2, en2,
               wn3, bn3, gn3, en3, wtc, btc, wbc, enc_ref, p_ref, q_ref):
    x = x_ref[...]
    vol = vol_ref[...]
    f32 = jnp.float32
    encs = []
    for m in range(2):
        h = jnp.dot(x, wn1[m], preferred_element_type=f32) + bn1[m]
        h = jax.nn.relu(_ln_norm(h) * gn1[m] + en1[m])
        h = jnp.dot(h, wn2[m], preferred_element_type=f32) + bn2[m]
        h = jax.nn.relu(_ln_norm(h) * gn2[m] + en2[m])
        h = jnp.dot(h, wn3[m], preferred_element_type=f32) + bn3[m]
        h = jnp.tanh(_ln_norm(h) * gn3[m] + en3[m])
        encs.append(h)
    enc = jnp.where(vol == 1.0, encs[1], encs[0])
    enc_ref[...] = enc
    p_ref[...] = jnp.dot(enc, wtc[0], preferred_element_type=f32) + btc[0]
    q_ref[...] = jnp.dot(enc, wbc[0], preferred_element_type=f32)


def _sc_body(p_hbm, q_hbm, s_hbm, e_hbm, vol_hbm, h1_hbm, eid_hbm,
             vol_v, sidx, eidx, rp, rq, eidv, prow, qrow, sem_p, sem_q):
    wid = lax.axis_index("s") * 2 + lax.axis_index("c")
    pltpu.sync_copy(vol_hbm, vol_v)
    nch = (N_CHUNKS // NW) + jnp.where(wid < (N_CHUNKS % NW), 1, 0)

    def chunk(it, carry):
        base = (wid + it * NW) * CH
        pltpu.sync_copy(s_hbm.at[pl.ds(base, CH)], sidx)
        pltpu.sync_copy(e_hbm.at[pl.ds(base, CH)], eidx)
        for g in range(CH // 16):
            sl = pl.ds(g * 16, 16)
            s16 = sidx[sl]
            e16 = eidx[sl]
            vs = plsc.load_gather(vol_v, [s16])
            ve = plsc.load_gather(vol_v, [e16])
            ex = vs + ve
            eid = jnp.where((vs == 1) & (ve == 0), 3, ex)
            eidv[sl] = eid.astype(jnp.float32)
            rp[sl] = s16 * 3 + ex
            rq[sl] = e16 * 3 + ex
        cp = pltpu.async_copy(p_hbm.at[rp], prow, sem_p)
        cq = pltpu.async_copy(q_hbm.at[rq], qrow, sem_q)
        cp.wait()
        cq.wait()

        def addrow(r, c2):
            for c in range(HIDDEN // 16):
                cs = pl.ds(c * 16, 16)
                prow[r, cs] = prow[r, cs] + qrow[r, cs]
            return c2
        lax.fori_loop(0, CH, addrow, 0)
        pltpu.sync_copy(prow, h1_hbm.at[pl.ds(base, CH)])
        pltpu.sync_copy(eidv, eid_hbm.at[pl.ds(base, CH)])
        return carry
    lax.fori_loop(0, nch, chunk, 0)


def _sel(m0, m1, m2, p):
    return m0 * p[0] + m1 * p[1] + m2 * p[2]


def _edge_body(h1_ref, eid_ref, g1, e1, w2c, b2, g2, e2, w3c, b3, g3, e3, out_ref):
    h1 = h1_ref[...]
    eid = eid_ref[...]
    f32 = jnp.float32
    m0 = (eid == 0.0).astype(f32)
    m1 = (eid == 1.0).astype(f32)
    m2 = (eid == 2.0).astype(f32)
    xh = _ln_norm(h1)
    a1 = jax.nn.relu(xh * _sel(m0, m1, m2, g1) + _sel(m0, m1, m2, e1))
    a1c = jnp.concatenate([a1 * m0, a1 * m1, a1 * m2], axis=-1)
    z2 = _bdot(a1c, w2c[0]) + _sel(m0, m1, m2, b2)
    a2 = jax.nn.relu(_ln_norm(z2) * _sel(m0, m1, m2, g2) + _sel(m0, m1, m2, e2))
    a2c = jnp.concatenate([a2 * m0, a2 * m1, a2 * m2], axis=-1)
    z3 = _bdot(a2c, w3c[0]) + _sel(m0, m1, m2, b3)
    y = jnp.tanh(_ln_norm(z3) * _sel(m0, m1, m2, g3) + _sel(m0, m1, m2, e3))
    out_ref[...] = y * (m0 + m1 + m2)


def _full(shape):
    return pl.BlockSpec(shape, lambda i: (0,) * len(shape))


def kernel(x, node_params, edge_params, edge_index, volume_id):
    f32 = jnp.float32
    bf16 = jnp.bfloat16
    w0 = jnp.concatenate([node_params[0][0][0], jnp.zeros((4, HIDDEN), f32)], axis=0)
    wn1 = jnp.stack([w0, node_params[1][0][0]])
    bn1 = jnp.stack([node_params[0][0][1], node_params[1][0][1]])
    gn1 = jnp.stack([node_params[0][0][2], node_params[1][0][2]])
    en1 = jnp.stack([node_params[0][0][3], node_params[1][0][3]])
    wn2 = jnp.stack([node_params[0][1][0], node_params[1][1][0]])
    bn2 = jnp.stack([node_params[0][1][1], node_params[1][1][1]])
    gn2 = jnp.stack([node_params[0][1][2], node_params[1][1][2]])
    en2 = jnp.stack([node_params[0][1][3], node_params[1][1][3]])
    wn3 = jnp.stack([node_params[0][2][0], node_params[1][2][0]])
    bn3 = jnp.stack([node_params[0][2][1], node_params[1][2][1]])
    gn3 = jnp.stack([node_params[0][2][2], node_params[1][2][2]])
    en3 = jnp.stack([node_params[0][2][3], node_params[1][2][3]])

    wtc = jnp.concatenate([edge_params[e][0][0][:HIDDEN] for e in range(3)], axis=1)[None]
    wbc = jnp.concatenate([edge_params[e][0][0][HIDDEN:] for e in range(3)], axis=1)[None]
    btc = jnp.concatenate([edge_params[e][0][1] for e in range(3)], axis=0)[None]
    g1 = jnp.stack([edge_params[e][0][2] for e in range(3)])
    e1 = jnp.stack([edge_params[e][0][3] for e in range(3)])
    w2c = jnp.concatenate([edge_params[e][1][0] for e in range(3)], axis=0)[None].astype(bf16)
    b2 = jnp.stack([edge_params[e][1][1] for e in range(3)])
    g2 = jnp.stack([edge_params[e][1][2] for e in range(3)])
    e2 = jnp.stack([edge_params[e][1][3] for e in range(3)])
    w3c = jnp.concatenate([edge_params[e][2][0] for e in range(3)], axis=0)[None].astype(bf16)
    b3 = jnp.stack([edge_params[e][2][1] for e in range(3)])
    g3 = jnp.stack([edge_params[e][2][2] for e in range(3)])
    e3 = jnp.stack([edge_params[e][2][3] for e in range(3)])

    volf = volume_id.astype(f32)[:, None]
    voli = volume_id.astype(jnp.int32)
    start = edge_index[0].astype(jnp.int32)
    end = edge_index[1].astype(jnp.int32)

    n_blocks = N_NODES // NB_NODE
    enc, p_cat, q_cat = pl.pallas_call(
        _node_body,
        grid=(n_blocks,),
        in_specs=[
            pl.BlockSpec((NB_NODE, 12), lambda i: (i, 0)),
            pl.BlockSpec((NB_NODE, 1), lambda i: (i, 0)),
            _full((2, 12, HIDDEN)), _full((2, HIDDEN)), _full((2, HIDDEN)), _full((2, HIDDEN)),
            _full((2, HIDDEN, HIDDEN)), _full((2, HIDDEN)), _full((2, HIDDEN)), _full((2, HIDDEN)),
            _full((2, HIDDEN, HIDDEN)), _full((2, HIDDEN)), _full((2, HIDDEN)), _full((2, HIDDEN)),
            _full((1, HIDDEN, 3 * HIDDEN)), _full((1, 3 * HIDDEN)), _full((1, HIDDEN, 3 * HIDDEN)),
        ],
        out_specs=[
            pl.BlockSpec((NB_NODE, HIDDEN), lambda i: (i, 0)),
            pl.BlockSpec((NB_NODE, 3 * HIDDEN), lambda i: (i, 0)),
            pl.BlockSpec((NB_NODE, 3 * HIDDEN), lambda i: (i, 0)),
        ],
        out_shape=[
            jax.ShapeDtypeStruct((N_NODES, HIDDEN), f32),
            jax.ShapeDtypeStruct((N_NODES, 3 * HIDDEN), f32),
            jax.ShapeDtypeStruct((N_NODES, 3 * HIDDEN), f32),
        ],
    )(x, volf, wn1, bn1, gn1, en1, wn2, bn2, gn2, en2, wn3, bn3, gn3, en3, wtc, btc, wbc)

    p_flat = p_cat.reshape(3 * N_NODES, HIDDEN)
    q_flat = q_cat.reshape(3 * N_NODES, HIDDEN)

    mesh = plsc.VectorSubcoreMesh(core_axis_name="c", subcore_axis_name="s")
    h1, eidf = pl.kernel(
        _sc_body,
        out_type=[
            jax.ShapeDtypeStruct((N_EDGES, HIDDEN), f32),
            jax.ShapeDtypeStruct((N_EDGES,), f32),
        ],
        mesh=mesh,
        compiler_params=pltpu.CompilerParams(needs_layout_passes=False),
        scratch_types=[
            pltpu.VMEM((N_NODES,), jnp.int32),
            pltpu.VMEM((CH,), jnp.int32),
            pltpu.VMEM((CH,), jnp.int32),
            pltpu.VMEM((CH,), jnp.int32),
            pltpu.VMEM((CH,), jnp.int32),
            pltpu.VMEM((CH,), f32),
            pltpu.VMEM((CH, HIDDEN), f32),
            pltpu.VMEM((CH, HIDDEN), f32),
            pltpu.SemaphoreType.DMA,
            pltpu.SemaphoreType.DMA,
        ],
    )(p_flat, q_flat, start, end, voli)

    e_blocks = N_EDGES // NB_EDGE
    out_edges = pl.pallas_call(
        _edge_body,
        grid=(e_blocks,),
        in_specs=[
            pl.BlockSpec((NB_EDGE, HIDDEN), lambda i: (i, 0)),
            pl.BlockSpec((NB_EDGE, 1), lambda i: (i, 0)),
            _full((3, HIDDEN)), _full((3, HIDDEN)),
            _full((1, 3 * HIDDEN, HIDDEN)), _full((3, HIDDEN)), _full((3, HIDDEN)), _full((3, HIDDEN)),
            _full((1, 3 * HIDDEN, HIDDEN)), _full((3, HIDDEN)), _full((3, HIDDEN)), _full((3, HIDDEN)),
        ],
        out_specs=pl.BlockSpec((NB_EDGE, HIDDEN), lambda i: (i, 0)),
        out_shape=jax.ShapeDtypeStruct((N_EDGES, HIDDEN), f32),
    )(h1, eidf[:, None], g1, e1, w2c, b2, g2, e2, w3c, b3, g3, e3)

    return (enc, out_edges)
